# baseline (speedup 1.0000x reference)
import functools

import jax
import jax.numpy as jnp
from jax import lax
from jax.experimental import pallas as pl
from jax.experimental.pallas import tpu as pltpu
from jax.experimental.pallas import tpu_sc as plsc

NUM_ENTITIES = 1000000
EMBED_DIM = 64
BATCH = 16384


@functools.cache
def _build_gather():
    info = plsc.get_sparse_core_info()
    nc, ns = info.num_cores, info.num_subcores
    nw = nc * ns
    b_per_w = BATCH // nw

    mesh = plsc.VectorSubcoreMesh(core_axis_name="c", subcore_axis_name="s")

    @functools.partial(
        pl.kernel,
        mesh=mesh,
        out_type=jax.ShapeDtypeStruct((BATCH, EMBED_DIM), jnp.float32),
        scratch_types=[
            pltpu.VMEM((b_per_w,), jnp.int32),
            pltpu.VMEM((b_per_w, EMBED_DIM), jnp.float32),
            pltpu.SemaphoreType.DMA,
        ],
        compiler_params=pltpu.CompilerParams(use_tc_tiling_on_sc=False),
    )
    def gather(idx_hbm, table_hbm, out_hbm, idx_v, rows_v, sem):
        wid = lax.axis_index("s") * nc + lax.axis_index("c")
        base = wid * b_per_w
        pltpu.sync_copy(idx_hbm.at[pl.ds(base, b_per_w)], idx_v)
        pltpu.async_copy(table_hbm.at[idx_v], rows_v, sem).wait()
        pltpu.sync_copy(rows_v, out_hbm.at[pl.ds(base, b_per_w)])

    return gather


def kernel(entities, entity_embeddings):
    out = _build_gather()(entities.astype(jnp.int32), entity_embeddings)
    return out.reshape(-1, EMBED_DIM)


# speedup vs baseline: 2.5925x; 2.5925x over previous
import functools

import jax
import jax.numpy as jnp
from jax import lax
from jax.experimental import pallas as pl
from jax.experimental.pallas import tpu as pltpu
from jax.experimental.pallas import tpu_sc as plsc

NUM_ENTITIES = 1000000
EMBED_DIM = 64
BATCH = 16384
LANES = 128
NBUF = 4
GRP = 16


@functools.cache
def _build_gather():
    info = plsc.get_sparse_core_info()
    nc, ns = info.num_cores, info.num_subcores
    nw = nc * ns
    b_per_w = BATCH // nw
    n_grp = b_per_w // GRP

    mesh = plsc.VectorSubcoreMesh(core_axis_name="c", subcore_axis_name="s")

    @functools.partial(
        pl.kernel,
        mesh=mesh,
        out_type=jax.ShapeDtypeStruct((EMBED_DIM, BATCH), jnp.float32),
        scratch_types=[
            pltpu.VMEM((b_per_w,), jnp.int32),
            pltpu.VMEM((EMBED_DIM, b_per_w), jnp.float32),
        ]
        + [pltpu.VMEM((EMBED_DIM, LANES), jnp.float32) for _ in range(NBUF)]
        + [pltpu.SemaphoreType.DMA for _ in range(NBUF)],
        compiler_params=pltpu.CompilerParams(needs_layout_passes=False),
    )
    def gather(idx_hbm, table_t_hbm, out_hbm, idx_v, staging, *bufs_sems):
        bufs = bufs_sems[:NBUF]
        sems = bufs_sems[NBUF:]
        wid = lax.axis_index("s") * nc + lax.axis_index("c")
        base = wid * b_per_w
        pltpu.sync_copy(idx_hbm.at[pl.ds(base, b_per_w)], idx_v)

        lane_iota = lax.iota(jnp.int32, 16)

        def fire(r, b):
            t = pl.multiple_of((r >> 7) << 7, LANES)
            pltpu.make_async_copy(
                table_t_hbm.at[:, pl.ds(t, LANES)], bufs[b], sems[b]
            ).start()

        idx16_0 = idx_v[pl.ds(0, GRP)]
        for b in range(NBUF):
            fire(idx16_0[b], b)

        def group(g, idx16):
            nxt_off = pl.multiple_of(
                jnp.where(g == n_grp - 1, 0, (g + 1) * GRP), GRP
            )
            idx16_nxt = idx_v[pl.ds(nxt_off, GRP)]
            i0 = g * GRP
            for b in range(GRP):
                s = b % NBUF
                pltpu.make_async_copy(
                    table_t_hbm.at[:, pl.ds(0, LANES)], bufs[s], sems[s]
                ).wait()
                r = idx16[b]
                lane = jnp.full((16,), r & (LANES - 1), jnp.int32)
                col_idx = jnp.full((16,), i0 + b, jnp.int32)
                for k in range(EMBED_DIM // 16):
                    feat = lane_iota + (16 * k)
                    v = plsc.load_gather(bufs[s], [feat, lane])
                    plsc.store_scatter(staging, [feat, col_idx], v)
                if b + NBUF < GRP:
                    fire(idx16[b + NBUF], s)
                else:

                    @pl.when(g < n_grp - 1)
                    def _():
                        fire(idx16_nxt[b + NBUF - GRP], s)

            return idx16_nxt

        lax.fori_loop(0, n_grp, group, idx16_0)
        pltpu.sync_copy(staging, out_hbm.at[:, pl.ds(base, b_per_w)])

    return gather


def kernel(entities, entity_embeddings):
    out_t = _build_gather()(entities.astype(jnp.int32), entity_embeddings.T)
    return out_t.T


# speedup vs baseline: 3.0294x; 1.1685x over previous
import functools

import jax
import jax.numpy as jnp
from jax import lax
from jax.experimental import pallas as pl
from jax.experimental.pallas import tpu as pltpu
from jax.experimental.pallas import tpu_sc as plsc

NUM_ENTITIES = 1000000
EMBED_DIM = 64
BATCH = 16384
LANES = 128
NBUF = 8
GRP = 16


@functools.cache
def _build_gather():
    info = plsc.get_sparse_core_info()
    nc, ns = info.num_cores, info.num_subcores
    nw = nc * ns
    b_per_w = BATCH // nw
    n_grp = b_per_w // GRP

    mesh = plsc.VectorSubcoreMesh(core_axis_name="c", subcore_axis_name="s")

    @functools.partial(
        pl.kernel,
        mesh=mesh,
        out_type=jax.ShapeDtypeStruct((EMBED_DIM, BATCH), jnp.float32),
        scratch_types=[
            pltpu.VMEM((b_per_w,), jnp.int32),
            pltpu.VMEM((EMBED_DIM, b_per_w), jnp.float32),
        ]
        + [pltpu.VMEM((EMBED_DIM, LANES), jnp.float32) for _ in range(NBUF)]
        + [pltpu.SemaphoreType.DMA for _ in range(NBUF)],
        compiler_params=pltpu.CompilerParams(needs_layout_passes=False),
    )
    def gather(idx_hbm, table_t_hbm, out_hbm, idx_v, staging, *bufs_sems):
        bufs = bufs_sems[:NBUF]
        sems = bufs_sems[NBUF:]
        wid = lax.axis_index("s") * nc + lax.axis_index("c")
        base = wid * b_per_w
        pltpu.sync_copy(idx_hbm.at[pl.ds(base, b_per_w)], idx_v)

        lane_iota = lax.iota(jnp.int32, 16)

        def fire(r, b):
            t = pl.multiple_of((r >> 7) << 7, LANES)
            pltpu.make_async_copy(
                table_t_hbm.at[:, pl.ds(t, LANES)], bufs[b], sems[b]
            ).start()

        idx16_0 = idx_v[pl.ds(0, GRP)]
        for b in range(NBUF):
            fire(idx16_0[b], b)

        def group(g, idx16):
            nxt_off = pl.multiple_of(
                jnp.where(g == n_grp - 1, 0, (g + 1) * GRP), GRP
            )
            idx16_nxt = idx_v[pl.ds(nxt_off, GRP)]
            i0 = g * GRP
            for b in range(GRP):
                s = b % NBUF
                pltpu.make_async_copy(
                    table_t_hbm.at[:, pl.ds(0, LANES)], bufs[s], sems[s]
                ).wait()
                r = idx16[b]
                lane = jnp.full((16,), r & (LANES - 1), jnp.int32)
                col_idx = jnp.full((16,), i0 + b, jnp.int32)
                for k in range(EMBED_DIM // 16):
                    feat = lane_iota + (16 * k)
                    v = plsc.load_gather(bufs[s], [feat, lane])
                    plsc.store_scatter(staging, [feat, col_idx], v)
                if b + NBUF < GRP:
                    fire(idx16[b + NBUF], s)
                else:

                    @pl.when(g < n_grp - 1)
                    def _():
                        fire(idx16_nxt[b + NBUF - GRP], s)

            return idx16_nxt

        lax.fori_loop(0, n_grp, group, idx16_0)
        pltpu.sync_copy(staging, out_hbm.at[:, pl.ds(base, b_per_w)])

    return gather


def kernel(entities, entity_embeddings):
    out_t = _build_gather()(entities.astype(jnp.int32), entity_embeddings.T)
    return out_t.T
